# Initial kernel scaffold; baseline (speedup 1.0000x reference)
import jax
import jax.numpy as jnp
from jax.experimental import pallas as pl


def kernel(x, W_enc, b_enc, W_dec, b_dec):
    raise NotImplementedError("write your pallas kernel here")


# speedup vs baseline: 17.0761x; 17.0761x over previous
import functools

import jax
import jax.numpy as jnp
from jax.experimental import pallas as pl
from jax.experimental.pallas import tpu as pltpu

_K = 100
_BM = 256


def _body(x_ref, we_ref, be_ref, wd_ref, bd_ref, o_ref):
    x = x_ref[...] - bd_ref[...]
    pre = jnp.dot(x, we_ref[...], preferred_element_type=jnp.float32)
    y = jnp.maximum(pre + be_ref[...], 0.0)
    bits = jax.lax.bitcast_convert_type(y, jnp.int32)

    def step(i, t):
        cand = jnp.bitwise_or(t, jax.lax.shift_left(1, 30 - i))
        cnt = jnp.sum((bits >= cand).astype(jnp.float32), axis=1, keepdims=True)
        return jnp.where(cnt >= float(_K), cand, t)

    t = jax.lax.fori_loop(0, 31, step, jnp.zeros((x.shape[0], 1), jnp.int32))
    enc = jnp.where(bits >= t, y, 0.0)
    o_ref[...] = jnp.dot(enc, wd_ref[...], preferred_element_type=jnp.float32) + bd_ref[...]


@jax.jit
def kernel(x, W_enc, b_enc, W_dec, b_dec):
    B, d_in = x.shape
    d_sae = W_enc.shape[1]
    be = b_enc.reshape(1, d_sae)
    bd = b_dec.reshape(1, d_in)
    grid = (B // _BM,)
    return pl.pallas_call(
        _body,
        grid=grid,
        in_specs=[
            pl.BlockSpec((_BM, d_in), lambda i: (i, 0)),
            pl.BlockSpec((d_in, d_sae), lambda i: (0, 0)),
            pl.BlockSpec((1, d_sae), lambda i: (0, 0)),
            pl.BlockSpec((d_sae, d_in), lambda i: (0, 0)),
            pl.BlockSpec((1, d_in), lambda i: (0, 0)),
        ],
        out_specs=pl.BlockSpec((_BM, d_in), lambda i: (i, 0)),
        out_shape=jax.ShapeDtypeStruct((B, d_in), jnp.float32),
    )(x, W_enc, be, W_dec, bd)
